# Initial kernel scaffold; baseline (speedup 1.0000x reference)
import jax
import jax.numpy as jnp
from jax.experimental import pallas as pl


def kernel(x, edge_index, W, b):
    raise NotImplementedError("write your pallas kernel here")


# speedup vs baseline: 38.4279x; 38.4279x over previous
import dataclasses
import functools

import jax
import jax.numpy as jnp
from jax import lax
from jax.experimental import pallas as pl
from jax.experimental.pallas import tpu as pltpu
from jax.experimental.pallas import tpu_sc as plsc

N_NODES = 10000
N_EDGES = 320000
F_IN = 128
F_OUT = 16

NC = 2
NS = 16
NW = NC * NS
L = 16

NBLK = 80
E_PAD = NW * NBLK * 128
NPAD = 10240
HROWS = NPAD // L
HSLICE = HROWS // NS
ASLICE = NPAD // NS
NIOTA = HROWS // 128

_mesh = plsc.VectorSubcoreMesh(core_axis_name="c", subcore_axis_name="s")

_sc_params = pltpu.CompilerParams()
for _field, _val in (("needs_layout_passes", False), ("use_tc_tiling_on_sc", False)):
    if _field in pltpu.CompilerParams.__dataclass_fields__:
        _sc_params = dataclasses.replace(_sc_params, **{_field: _val})


@functools.partial(
    pl.kernel,
    mesh=_mesh,
    out_type=jax.ShapeDtypeStruct((NC, HROWS, L), jnp.float32),
    scratch_types=[
        pltpu.VMEM((NBLK, 1, 128), jnp.int32),
        pltpu.VMEM((HROWS, L), jnp.float32),
        pltpu.VMEM((NIOTA, 1, 128), jnp.int32),
        pltpu.VMEM_SHARED((HROWS, L), jnp.float32),
    ],
    compiler_params=_sc_params,
)
def _hist_kernel(dst_hbm, iota_hbm, out_hbm, dst_v, hist_v, iota_v, hist_sh):
    c = lax.axis_index("c")
    s = lax.axis_index("s")
    wid = c * NS + s
    pltpu.sync_copy(dst_hbm.at[wid], dst_v)
    pltpu.sync_copy(iota_hbm, iota_v)

    zeros = jnp.zeros((L,), jnp.float32)

    @pl.loop(0, HROWS)
    def _(r):
        hist_v[r, :] = zeros

    pltpu.sync_copy(
        hist_v.at[pl.ds(s * HSLICE, HSLICE)],
        hist_sh.at[pl.ds(s * HSLICE, HSLICE)],
    )
    plsc.subcore_barrier()

    ones = jnp.ones((L,), jnp.float32)

    @pl.loop(0, NBLK)
    def _(j):
        for k in range(128 // L):
            idx = dst_v[j, 0, pl.ds(k * L, L)]
            row = lax.shift_right_logical(idx, 4)
            col = lax.bitwise_and(idx, 15)
            plsc.addupdate_scatter(hist_v, [row, col], ones)

    @pl.loop(0, NIOTA)
    def _(r):
        pltpu.sync_copy(
            hist_v.at[pl.ds(r * 128, 128)],
            hist_sh.at[iota_v.at[r, 0]],
            add=True,
        )

    plsc.subcore_barrier()
    pltpu.sync_copy(
        hist_sh.at[pl.ds(s * HSLICE, HSLICE)],
        out_hbm.at[c, pl.ds(s * HSLICE, HSLICE)],
    )


@functools.partial(
    pl.kernel,
    mesh=_mesh,
    out_type=jax.ShapeDtypeStruct((NC, NPAD, F_OUT), jnp.float32),
    scratch_types=[
        pltpu.VMEM((NBLK, 1, 128), jnp.int32),
        pltpu.VMEM((NBLK, 1, 128), jnp.int32),
        pltpu.VMEM((128, F_OUT), jnp.float32),
        pltpu.VMEM_SHARED((NPAD, F_OUT), jnp.float32),
    ],
    compiler_params=_sc_params,
)
def _gather_scatter_kernel(
    h2_hbm, src_hbm, dsti_hbm, zeros_hbm, out_hbm, src_v, dst_v, msg_v, acc_sh
):
    c = lax.axis_index("c")
    s = lax.axis_index("s")
    wid = c * NS + s
    pltpu.sync_copy(
        zeros_hbm.at[pl.ds(s * ASLICE, ASLICE)],
        acc_sh.at[pl.ds(s * ASLICE, ASLICE)],
    )
    pltpu.sync_copy(src_hbm.at[wid], src_v)
    pltpu.sync_copy(dsti_hbm.at[wid], dst_v)
    plsc.subcore_barrier()

    @pl.loop(0, NBLK)
    def _(j):
        pltpu.sync_copy(h2_hbm.at[src_v.at[j, 0]], msg_v)
        pltpu.sync_copy(msg_v, acc_sh.at[dst_v.at[j, 0]], add=True)

    plsc.subcore_barrier()
    pltpu.sync_copy(
        acc_sh.at[pl.ds(s * ASLICE, ASLICE)],
        out_hbm.at[c, pl.ds(s * ASLICE, ASLICE)],
    )


def _matmul_body(x_ref, w_ref, o_ref):
    o_ref[...] = jnp.dot(x_ref[...], w_ref[...], preferred_element_type=jnp.float32)


def _scale_body(h_ref, hist_ref, h2_ref, dis_ref):
    deg = hist_ref[...][:, 0:1] + hist_ref[...][:, 1:2] + 1.0
    dis = lax.rsqrt(deg)
    h2_ref[...] = h_ref[...] * dis
    dis_ref[...] = jnp.broadcast_to(dis, (NPAD, F_OUT))


def _final_body(acc_ref, h2_ref, dis_ref, b_ref, o_ref):
    acc = acc_ref[0, :, :] + acc_ref[1, :, :]
    o_ref[...] = (acc + h2_ref[...]) * dis_ref[...] + b_ref[...]


def kernel(x, edge_index, W, b):
    src = edge_index[0]
    dst = edge_index[1]
    pad = jnp.full((E_PAD - N_EDGES,), N_NODES, dtype=jnp.int32)
    src_p = jnp.concatenate([src, pad]).reshape(NW, NBLK, 1, 128)
    dst_p = jnp.concatenate([dst, pad]).reshape(NW, NBLK, 1, 128)
    iota = jnp.arange(HROWS, dtype=jnp.int32).reshape(NIOTA, 1, 128)
    zeros = jnp.zeros((NPAD, F_OUT), jnp.float32)
    x_pad = jnp.zeros((NPAD, F_IN), jnp.float32).at[:N_NODES].set(x)

    hist = _hist_kernel(dst_p, iota)
    h = pl.pallas_call(
        _matmul_body,
        out_shape=jax.ShapeDtypeStruct((NPAD, F_OUT), jnp.float32),
    )(x_pad, W)

    hist_t = hist.reshape(NC, NPAD).T

    h2, dis_b = pl.pallas_call(
        _scale_body,
        out_shape=(
            jax.ShapeDtypeStruct((NPAD, F_OUT), jnp.float32),
            jax.ShapeDtypeStruct((NPAD, F_OUT), jnp.float32),
        ),
    )(h, hist_t)

    acc = _gather_scatter_kernel(h2, src_p, dst_p, zeros)

    out = pl.pallas_call(
        _final_body,
        out_shape=jax.ShapeDtypeStruct((NPAD, F_OUT), jnp.float32),
    )(acc, h2, dis_b, b.reshape(1, F_OUT))

    return out[:N_NODES]


# speedup vs baseline: 44.1449x; 1.1488x over previous
import dataclasses
import functools

import jax
import jax.numpy as jnp
from jax import lax
from jax.experimental import pallas as pl
from jax.experimental.pallas import tpu as pltpu
from jax.experimental.pallas import tpu_sc as plsc

N_NODES = 10000
N_EDGES = 320000
F_IN = 128
F_OUT = 16

NC = 2
NS = 16
L = 16

NB0 = 88
NB1 = 68
NREM = 4
E0 = NS * NB0 * 128
E1 = NS * NB1 * 128
NBMAX = NB0 + 1

NPAD = 10240
HROWS = NPAD // L
HSLICE = HROWS // NS
ASLICE = NPAD // NS
NIOTA = HROWS // 128

_mesh = plsc.VectorSubcoreMesh(core_axis_name="c", subcore_axis_name="s")

_sc_params = pltpu.CompilerParams()
for _field, _val in (("needs_layout_passes", False), ("use_tc_tiling_on_sc", False)):
    if _field in pltpu.CompilerParams.__dataclass_fields__:
        _sc_params = dataclasses.replace(_sc_params, **{_field: _val})


def _load_my_blocks(c, s, e0_hbm, e1_hbm, rem_hbm, idx_v):
    @pl.when(c == 0)
    def _():
        pltpu.sync_copy(e0_hbm.at[s], idx_v.at[pl.ds(0, NB0)])

    @pl.when(c == 1)
    def _():
        pltpu.sync_copy(e1_hbm.at[s], idx_v.at[pl.ds(0, NB1)])

    @pl.when((c == 0) & (s < NREM))
    def _():
        pltpu.sync_copy(rem_hbm.at[s], idx_v.at[NB0])

    return jnp.where(
        c == 0, jnp.where(s < NREM, NB0 + 1, NB0), NB1
    ).astype(jnp.int32)


@functools.partial(
    pl.kernel,
    mesh=_mesh,
    out_type=jax.ShapeDtypeStruct((NC, HROWS, L), jnp.float32),
    scratch_types=[
        pltpu.VMEM((NBMAX, 1, 128), jnp.int32),
        pltpu.VMEM((HROWS, L), jnp.float32),
        pltpu.VMEM((NIOTA, 1, 128), jnp.int32),
        pltpu.VMEM_SHARED((HROWS, L), jnp.float32),
    ],
    compiler_params=_sc_params,
)
def _hist_kernel(e0_hbm, e1_hbm, rem_hbm, iota_hbm, out_hbm, dst_v, hist_v, iota_v, hist_sh):
    c = lax.axis_index("c")
    s = lax.axis_index("s")
    nblk = _load_my_blocks(c, s, e0_hbm, e1_hbm, rem_hbm, dst_v)
    pltpu.sync_copy(iota_hbm, iota_v)

    zeros = jnp.zeros((L,), jnp.float32)

    @pl.loop(0, HROWS)
    def _(r):
        hist_v[r, :] = zeros

    pltpu.sync_copy(
        hist_v.at[pl.ds(s * HSLICE, HSLICE)],
        hist_sh.at[pl.ds(s * HSLICE, HSLICE)],
    )
    plsc.subcore_barrier()

    ones = jnp.ones((L,), jnp.float32)

    @pl.loop(0, nblk)
    def _(j):
        for k in range(128 // L):
            idx = dst_v[j, 0, pl.ds(k * L, L)]
            row = lax.shift_right_logical(idx, 4)
            col = lax.bitwise_and(idx, 15)
            plsc.addupdate_scatter(hist_v, [row, col], ones)

    @pl.loop(0, NIOTA)
    def _(r):
        pltpu.sync_copy(
            hist_v.at[pl.ds(r * 128, 128)],
            hist_sh.at[iota_v.at[r, 0]],
            add=True,
        )

    plsc.subcore_barrier()
    pltpu.sync_copy(
        hist_sh.at[pl.ds(s * HSLICE, HSLICE)],
        out_hbm.at[c, pl.ds(s * HSLICE, HSLICE)],
    )


@functools.partial(
    pl.kernel,
    mesh=_mesh,
    out_type=jax.ShapeDtypeStruct((NC, NPAD, F_OUT), jnp.float32),
    scratch_types=[
        pltpu.VMEM((NBMAX, 1, 128), jnp.int32),
        pltpu.VMEM((NBMAX, 1, 128), jnp.int32),
        pltpu.VMEM((128, F_OUT), jnp.float32),
        pltpu.VMEM_SHARED((NPAD, F_OUT), jnp.float32),
    ],
    compiler_params=_sc_params,
)
def _gather_scatter_kernel(
    h2_hbm, es0_hbm, es1_hbm, srem_hbm, ed0_hbm, ed1_hbm, drem_hbm,
    zeros_hbm, out_hbm, src_v, dst_v, msg_v, acc_sh
):
    c = lax.axis_index("c")
    s = lax.axis_index("s")
    pltpu.sync_copy(
        zeros_hbm.at[pl.ds(s * ASLICE, ASLICE)],
        acc_sh.at[pl.ds(s * ASLICE, ASLICE)],
    )
    nblk = _load_my_blocks(c, s, es0_hbm, es1_hbm, srem_hbm, src_v)
    _load_my_blocks(c, s, ed0_hbm, ed1_hbm, drem_hbm, dst_v)
    plsc.subcore_barrier()

    @pl.loop(0, nblk)
    def _(j):
        pltpu.sync_copy(h2_hbm.at[src_v.at[j, 0]], msg_v)
        pltpu.sync_copy(msg_v, acc_sh.at[dst_v.at[j, 0]], add=True)

    plsc.subcore_barrier()
    pltpu.sync_copy(
        acc_sh.at[pl.ds(s * ASLICE, ASLICE)],
        out_hbm.at[c, pl.ds(s * ASLICE, ASLICE)],
    )


def _matmul_body(x_ref, w_ref, o_ref):
    o_ref[pl.ds(0, N_NODES), :] = jnp.dot(
        x_ref[...], w_ref[...], preferred_element_type=jnp.float32
    )


def _scale_body(h_ref, histt_ref, h2_ref):
    ht = histt_ref[...]
    dis = lax.rsqrt(ht[:, 0:1] + ht[:, 1:2] + 1.0)
    h2_ref[...] = h_ref[...] * dis


def _final_body(acc_ref, h2_ref, histt_ref, b_ref, o_ref):
    ht = histt_ref[...]
    dis = lax.rsqrt(ht[:, 0:1] + ht[:, 1:2] + 1.0)
    acc = acc_ref[0] + acc_ref[1]
    o_ref[...] = (acc + h2_ref[...]) * dis + b_ref[...]


def kernel(x, edge_index, W, b):
    e_main0 = edge_index[:, :E0].reshape(2, NS, NB0, 1, 128)
    e_main1 = edge_index[:, E0:E0 + E1].reshape(2, NS, NB1, 1, 128)
    e_rem = edge_index[:, E0 + E1:].reshape(2, NREM, 1, 128)
    iota = jnp.arange(HROWS, dtype=jnp.int32).reshape(NIOTA, 1, 128)
    zeros = jnp.zeros((NPAD, F_OUT), jnp.float32)

    hist = _hist_kernel(e_main0[1], e_main1[1], e_rem[1], iota)
    h = pl.pallas_call(
        _matmul_body,
        out_shape=jax.ShapeDtypeStruct((NPAD, F_OUT), jnp.float32),
    )(x, W)

    hist_t = hist.reshape(NC, NPAD).T

    h2 = pl.pallas_call(
        _scale_body,
        out_shape=jax.ShapeDtypeStruct((NPAD, F_OUT), jnp.float32),
    )(h, hist_t)

    acc = _gather_scatter_kernel(
        h2, e_main0[0], e_main1[0], e_rem[0], e_main0[1], e_main1[1], e_rem[1], zeros
    )

    BODY = 1000
    out = pl.pallas_call(
        _final_body,
        grid=(N_NODES // BODY,),
        in_specs=[
            pl.BlockSpec((NC, BODY, F_OUT), lambda i: (0, i, 0)),
            pl.BlockSpec((BODY, F_OUT), lambda i: (i, 0)),
            pl.BlockSpec((BODY, 2), lambda i: (i, 0)),
            pl.BlockSpec((1, F_OUT), lambda i: (0, 0)),
        ],
        out_specs=pl.BlockSpec((BODY, F_OUT), lambda i: (i, 0)),
        out_shape=jax.ShapeDtypeStruct((N_NODES, F_OUT), jnp.float32),
    )(acc, h2, hist_t, b.reshape(1, F_OUT))

    return out


# speedup vs baseline: 61.8389x; 1.4008x over previous
import dataclasses
import functools

import jax
import jax.numpy as jnp
from jax import lax
from jax.experimental import pallas as pl
from jax.experimental.pallas import tpu as pltpu
from jax.experimental.pallas import tpu_sc as plsc

N_NODES = 10000
N_EDGES = 320000
F_IN = 128
F_OUT = 16

NC = 2
NS = 16
L = 16

NB0 = 88
NB1 = 68
NREM = 4
E0 = NS * NB0 * 128
E1 = NS * NB1 * 128
KI = 4
KW = KI * 128
NM0 = NB0 // KI
NM1 = NB1 // KI
NMMAX = NM0 + 1

NPAD = 10240
HROWS = NPAD // L
HSLICE = HROWS // NS
ASLICE = NPAD // NS
NIOTA = HROWS // 128

_mesh = plsc.VectorSubcoreMesh(core_axis_name="c", subcore_axis_name="s")

_sc_params = pltpu.CompilerParams()
for _field, _val in (("needs_layout_passes", False), ("use_tc_tiling_on_sc", False)):
    if _field in pltpu.CompilerParams.__dataclass_fields__:
        _sc_params = dataclasses.replace(_sc_params, **{_field: _val})


def _load_my_blocks(c, s, e0_hbm, e1_hbm, rem_hbm, idx_v):
    @pl.when(c == 0)
    def _():
        pltpu.sync_copy(e0_hbm.at[s], idx_v.at[pl.ds(0, NM0 * KW)])

    @pl.when(c == 1)
    def _():
        pltpu.sync_copy(e1_hbm.at[s], idx_v.at[pl.ds(0, NM1 * KW)])

    @pl.when((c == 0) & (s < NREM))
    def _():
        pltpu.sync_copy(rem_hbm.at[s], idx_v.at[pl.ds(NM0 * KW, 128)])


@functools.partial(
    pl.kernel,
    mesh=_mesh,
    out_type=jax.ShapeDtypeStruct((NC, HROWS, L), jnp.float32),
    scratch_types=[
        pltpu.VMEM((NMMAX * KW,), jnp.int32),
        pltpu.VMEM((HROWS, L), jnp.float32),
        pltpu.VMEM((NIOTA, 128), jnp.int32),
        pltpu.VMEM_SHARED((HROWS, L), jnp.float32),
    ],
    compiler_params=_sc_params,
)
def _hist_kernel(e0_hbm, e1_hbm, rem_hbm, iota_hbm, out_hbm, dst_v, hist_v, iota_v, hist_sh):
    c = lax.axis_index("c")
    s = lax.axis_index("s")
    _load_my_blocks(c, s, e0_hbm, e1_hbm, rem_hbm, dst_v)
    pltpu.sync_copy(iota_hbm, iota_v)
    nmac = jnp.where(c == 0, NM0, NM1).astype(jnp.int32)

    zeros = jnp.zeros((L,), jnp.float32)

    @pl.loop(0, HROWS)
    def _(r):
        hist_v[r, :] = zeros

    pltpu.sync_copy(
        hist_v.at[pl.ds(s * HSLICE, HSLICE)],
        hist_sh.at[pl.ds(s * HSLICE, HSLICE)],
    )
    plsc.subcore_barrier()

    ones = jnp.ones((L,), jnp.float32)

    def _count16(j, k):
        idx = dst_v[pl.ds(j * KW + k * L, L)]
        row = lax.shift_right_logical(idx, 4)
        col = lax.bitwise_and(idx, 15)
        plsc.addupdate_scatter(hist_v, [row, col], ones)

    @pl.loop(0, nmac)
    def _(j):
        for k in range(KW // L):
            _count16(j, k)

    @pl.when((c == 0) & (s < NREM))
    def _():
        for k in range(128 // L):
            _count16(NM0, k)

    @pl.loop(0, NIOTA)
    def _(r):
        pltpu.sync_copy(
            hist_v.at[pl.ds(r * 128, 128)],
            hist_sh.at[iota_v.at[r]],
            add=True,
        )

    plsc.subcore_barrier()
    pltpu.sync_copy(
        hist_sh.at[pl.ds(s * HSLICE, HSLICE)],
        out_hbm.at[c, pl.ds(s * HSLICE, HSLICE)],
    )


@functools.partial(
    pl.kernel,
    mesh=_mesh,
    out_type=jax.ShapeDtypeStruct((NC, NPAD, F_OUT), jnp.float32),
    scratch_types=[
        pltpu.VMEM((NMMAX * KW,), jnp.int32),
        pltpu.VMEM((NMMAX * KW,), jnp.int32),
        pltpu.VMEM((KW, F_OUT), jnp.float32),
        pltpu.VMEM_SHARED((NPAD, F_OUT), jnp.float32),
    ],
    compiler_params=_sc_params,
)
def _gather_scatter_kernel(
    h2_hbm, es0_hbm, es1_hbm, srem_hbm, ed0_hbm, ed1_hbm, drem_hbm,
    zeros_hbm, out_hbm, src_v, dst_v, msg_v, acc_sh
):
    c = lax.axis_index("c")
    s = lax.axis_index("s")
    pltpu.sync_copy(
        zeros_hbm.at[pl.ds(s * ASLICE, ASLICE)],
        acc_sh.at[pl.ds(s * ASLICE, ASLICE)],
    )
    _load_my_blocks(c, s, es0_hbm, es1_hbm, srem_hbm, src_v)
    _load_my_blocks(c, s, ed0_hbm, ed1_hbm, drem_hbm, dst_v)
    plsc.subcore_barrier()

    nmac = jnp.where(c == 0, NM0, NM1).astype(jnp.int32)

    @pl.loop(0, nmac)
    def _(j):
        pltpu.sync_copy(h2_hbm.at[src_v.at[pl.ds(j * KW, KW)]], msg_v)
        pltpu.sync_copy(msg_v, acc_sh.at[dst_v.at[pl.ds(j * KW, KW)]], add=True)

    @pl.when((c == 0) & (s < NREM))
    def _():
        pltpu.sync_copy(
            h2_hbm.at[src_v.at[pl.ds(NM0 * KW, 128)]],
            msg_v.at[pl.ds(0, 128)],
        )
        pltpu.sync_copy(
            msg_v.at[pl.ds(0, 128)],
            acc_sh.at[dst_v.at[pl.ds(NM0 * KW, 128)]],
            add=True,
        )

    plsc.subcore_barrier()
    pltpu.sync_copy(
        acc_sh.at[pl.ds(s * ASLICE, ASLICE)],
        out_hbm.at[c, pl.ds(s * ASLICE, ASLICE)],
    )


def _matmul_body(x_ref, w_ref, o_ref):
    o_ref[pl.ds(0, N_NODES), :] = jnp.dot(
        x_ref[...], w_ref[...], preferred_element_type=jnp.float32
    )


def _scale_body(h_ref, histt_ref, h2_ref):
    ht = histt_ref[...]
    dis = lax.rsqrt(ht[:, 0:1] + ht[:, 1:2] + 1.0)
    h2_ref[...] = h_ref[...] * dis


def _final_body(acc_ref, h2_ref, histt_ref, b_ref, o_ref):
    ht = histt_ref[...]
    dis = lax.rsqrt(ht[:, 0:1] + ht[:, 1:2] + 1.0)
    acc = acc_ref[0] + acc_ref[1]
    o_ref[...] = (acc + h2_ref[...]) * dis + b_ref[...]


def kernel(x, edge_index, W, b):
    e_main0 = edge_index[:, :E0].reshape(2, NS, NM0 * KW)
    e_main1 = edge_index[:, E0:E0 + E1].reshape(2, NS, NM1 * KW)
    e_rem = edge_index[:, E0 + E1:].reshape(2, NREM, 128)
    iota = jnp.arange(HROWS, dtype=jnp.int32).reshape(NIOTA, 128)
    zeros = jnp.zeros((NPAD, F_OUT), jnp.float32)

    hist = _hist_kernel(e_main0[1], e_main1[1], e_rem[1], iota)
    h = pl.pallas_call(
        _matmul_body,
        out_shape=jax.ShapeDtypeStruct((NPAD, F_OUT), jnp.float32),
    )(x, W)

    hist_t = hist.reshape(NC, NPAD).T

    h2 = pl.pallas_call(
        _scale_body,
        out_shape=jax.ShapeDtypeStruct((NPAD, F_OUT), jnp.float32),
    )(h, hist_t)

    acc = _gather_scatter_kernel(
        h2, e_main0[0], e_main1[0], e_rem[0], e_main0[1], e_main1[1], e_rem[1], zeros
    )

    BODY = 1000
    out = pl.pallas_call(
        _final_body,
        grid=(N_NODES // BODY,),
        in_specs=[
            pl.BlockSpec((NC, BODY, F_OUT), lambda i: (0, i, 0)),
            pl.BlockSpec((BODY, F_OUT), lambda i: (i, 0)),
            pl.BlockSpec((BODY, 2), lambda i: (i, 0)),
            pl.BlockSpec((1, F_OUT), lambda i: (0, 0)),
        ],
        out_specs=pl.BlockSpec((BODY, F_OUT), lambda i: (i, 0)),
        out_shape=jax.ShapeDtypeStruct((N_NODES, F_OUT), jnp.float32),
    )(acc, h2, hist_t, b.reshape(1, F_OUT))

    return out


# speedup vs baseline: 63.4931x; 1.0267x over previous
import dataclasses
import functools

import jax
import jax.numpy as jnp
from jax import lax
from jax.experimental import pallas as pl
from jax.experimental.pallas import tpu as pltpu
from jax.experimental.pallas import tpu_sc as plsc

N_NODES = 10000
N_EDGES = 320000
F_IN = 128
F_OUT = 16

NC = 2
NS = 16
L = 16

NB0 = 80
NB1 = 76
NREM = 4
E0 = NS * NB0 * 128
E1 = NS * NB1 * 128
KI = 4
KW = KI * 128
NM0 = NB0 // KI
NM1 = NB1 // KI
NMMAX = NM0 + 1

NPAD = 10240
HROWS = NPAD // L
HSLICE = HROWS // NS
ASLICE = NPAD // NS
NIOTA = HROWS // 128

_mesh = plsc.VectorSubcoreMesh(core_axis_name="c", subcore_axis_name="s")

_sc_params = pltpu.CompilerParams()
for _field, _val in (("needs_layout_passes", False), ("use_tc_tiling_on_sc", False)):
    if _field in pltpu.CompilerParams.__dataclass_fields__:
        _sc_params = dataclasses.replace(_sc_params, **{_field: _val})


def _load_my_blocks(c, s, e0_hbm, e1_hbm, rem_hbm, idx_v):
    @pl.when(c == 0)
    def _():
        pltpu.sync_copy(e0_hbm.at[s], idx_v.at[pl.ds(0, NM0 * KW)])

    @pl.when(c == 1)
    def _():
        pltpu.sync_copy(e1_hbm.at[s], idx_v.at[pl.ds(0, NM1 * KW)])

    @pl.when((c == 0) & (s < NREM))
    def _():
        pltpu.sync_copy(rem_hbm.at[s], idx_v.at[pl.ds(NM0 * KW, 128)])


@functools.partial(
    pl.kernel,
    mesh=_mesh,
    out_type=jax.ShapeDtypeStruct((NC, HROWS, L), jnp.float32),
    scratch_types=[
        pltpu.VMEM((NMMAX * KW,), jnp.int32),
        pltpu.VMEM((HROWS, L), jnp.float32),
        pltpu.VMEM((NIOTA, 128), jnp.int32),
        pltpu.VMEM_SHARED((HROWS, L), jnp.float32),
    ],
    compiler_params=_sc_params,
)
def _hist_kernel(e0_hbm, e1_hbm, rem_hbm, iota_hbm, out_hbm, dst_v, hist_v, iota_v, hist_sh):
    c = lax.axis_index("c")
    s = lax.axis_index("s")
    _load_my_blocks(c, s, e0_hbm, e1_hbm, rem_hbm, dst_v)
    pltpu.sync_copy(iota_hbm, iota_v)
    nmac = jnp.where(c == 0, NM0, NM1).astype(jnp.int32)

    zeros = jnp.zeros((L,), jnp.float32)

    @pl.loop(0, HROWS)
    def _(r):
        hist_v[r, :] = zeros

    pltpu.sync_copy(
        hist_v.at[pl.ds(s * HSLICE, HSLICE)],
        hist_sh.at[pl.ds(s * HSLICE, HSLICE)],
    )
    plsc.subcore_barrier()

    ones = jnp.ones((L,), jnp.float32)

    def _count16(j, k):
        idx = dst_v[pl.ds(j * KW + k * L, L)]
        row = lax.shift_right_logical(idx, 4)
        col = lax.bitwise_and(idx, 15)
        plsc.addupdate_scatter(hist_v, [row, col], ones)

    @pl.loop(0, nmac)
    def _(j):
        for k in range(KW // L):
            _count16(j, k)

    @pl.when((c == 0) & (s < NREM))
    def _():
        for k in range(128 // L):
            _count16(NM0, k)

    @pl.loop(0, NIOTA)
    def _(r):
        pltpu.sync_copy(
            hist_v.at[pl.ds(r * 128, 128)],
            hist_sh.at[iota_v.at[r]],
            add=True,
        )

    plsc.subcore_barrier()
    pltpu.sync_copy(
        hist_sh.at[pl.ds(s * HSLICE, HSLICE)],
        out_hbm.at[c, pl.ds(s * HSLICE, HSLICE)],
    )


@functools.partial(
    pl.kernel,
    mesh=_mesh,
    out_type=jax.ShapeDtypeStruct((NC, NPAD, F_OUT), jnp.float32),
    scratch_types=[
        pltpu.VMEM((NMMAX * KW,), jnp.int32),
        pltpu.VMEM((NMMAX * KW,), jnp.int32),
        pltpu.VMEM((KW, F_OUT), jnp.float32),
        pltpu.VMEM_SHARED((NPAD, F_OUT), jnp.float32),
    ],
    compiler_params=_sc_params,
)
def _gather_scatter_kernel(
    h2_hbm, es0_hbm, es1_hbm, srem_hbm, ed0_hbm, ed1_hbm, drem_hbm,
    zeros_hbm, out_hbm, src_v, dst_v, msg_v, acc_sh
):
    c = lax.axis_index("c")
    s = lax.axis_index("s")
    pltpu.sync_copy(
        zeros_hbm.at[pl.ds(s * ASLICE, ASLICE)],
        acc_sh.at[pl.ds(s * ASLICE, ASLICE)],
    )
    _load_my_blocks(c, s, es0_hbm, es1_hbm, srem_hbm, src_v)
    _load_my_blocks(c, s, ed0_hbm, ed1_hbm, drem_hbm, dst_v)
    plsc.subcore_barrier()

    nmac = jnp.where(c == 0, NM0, NM1).astype(jnp.int32)

    @pl.loop(0, nmac)
    def _(j):
        pltpu.sync_copy(h2_hbm.at[src_v.at[pl.ds(j * KW, KW)]], msg_v)
        pltpu.sync_copy(msg_v, acc_sh.at[dst_v.at[pl.ds(j * KW, KW)]], add=True)

    @pl.when((c == 0) & (s < NREM))
    def _():
        pltpu.sync_copy(
            h2_hbm.at[src_v.at[pl.ds(NM0 * KW, 128)]],
            msg_v.at[pl.ds(0, 128)],
        )
        pltpu.sync_copy(
            msg_v.at[pl.ds(0, 128)],
            acc_sh.at[dst_v.at[pl.ds(NM0 * KW, 128)]],
            add=True,
        )

    plsc.subcore_barrier()
    pltpu.sync_copy(
        acc_sh.at[pl.ds(s * ASLICE, ASLICE)],
        out_hbm.at[c, pl.ds(s * ASLICE, ASLICE)],
    )


def _matmul_body(x_ref, w_ref, o_ref):
    o_ref[pl.ds(0, N_NODES), :] = jnp.dot(
        x_ref[...], w_ref[...], preferred_element_type=jnp.float32
    )


def _scale_body(h_ref, histt_ref, h2_ref):
    ht = histt_ref[...]
    dis = lax.rsqrt(ht[:, 0:1] + ht[:, 1:2] + 1.0)
    h2_ref[...] = h_ref[...] * dis


def _final_body(acc_ref, h2_ref, histt_ref, b_ref, o_ref):
    ht = histt_ref[...]
    dis = lax.rsqrt(ht[:, 0:1] + ht[:, 1:2] + 1.0)
    acc = acc_ref[0] + acc_ref[1]
    o_ref[...] = (acc + h2_ref[...]) * dis + b_ref[...]


def kernel(x, edge_index, W, b):
    e_main0 = edge_index[:, :E0].reshape(2, NS, NM0 * KW)
    e_main1 = edge_index[:, E0:E0 + E1].reshape(2, NS, NM1 * KW)
    e_rem = edge_index[:, E0 + E1:].reshape(2, NREM, 128)
    iota = jnp.arange(HROWS, dtype=jnp.int32).reshape(NIOTA, 128)
    zeros = jnp.zeros((NPAD, F_OUT), jnp.float32)

    hist = _hist_kernel(e_main0[1], e_main1[1], e_rem[1], iota)
    h = pl.pallas_call(
        _matmul_body,
        out_shape=jax.ShapeDtypeStruct((NPAD, F_OUT), jnp.float32),
    )(x, W)

    hist_t = hist.reshape(NC, NPAD).T

    h2 = pl.pallas_call(
        _scale_body,
        out_shape=jax.ShapeDtypeStruct((NPAD, F_OUT), jnp.float32),
    )(h, hist_t)

    acc = _gather_scatter_kernel(
        h2, e_main0[0], e_main1[0], e_rem[0], e_main0[1], e_main1[1], e_rem[1], zeros
    )

    BODY = 1000
    out = pl.pallas_call(
        _final_body,
        grid=(N_NODES // BODY,),
        in_specs=[
            pl.BlockSpec((NC, BODY, F_OUT), lambda i: (0, i, 0)),
            pl.BlockSpec((BODY, F_OUT), lambda i: (i, 0)),
            pl.BlockSpec((BODY, 2), lambda i: (i, 0)),
            pl.BlockSpec((1, F_OUT), lambda i: (0, 0)),
        ],
        out_specs=pl.BlockSpec((BODY, F_OUT), lambda i: (i, 0)),
        out_shape=jax.ShapeDtypeStruct((N_NODES, F_OUT), jnp.float32),
    )(acc, h2, hist_t, b.reshape(1, F_OUT))

    return out


# speedup vs baseline: 63.9489x; 1.0072x over previous
import dataclasses
import functools

import jax
import jax.numpy as jnp
from jax import lax
from jax.experimental import pallas as pl
from jax.experimental.pallas import tpu as pltpu
from jax.experimental.pallas import tpu_sc as plsc

N_NODES = 10000
N_EDGES = 320000
F_IN = 128
F_OUT = 16

NC = 2
NS = 16
L = 16

NB0 = 80
NB1 = 76
NREM = 4
E0 = NS * NB0 * 128
E1 = NS * NB1 * 128
KI = 4
KW = KI * 128
NM0 = NB0 // KI
NM1 = NB1 // KI
NMMAX = NM0 + 1

NPAD = 10240
HROWS = NPAD // L
HSLICE = HROWS // NS
ASLICE = N_NODES // NS
NIOTA = HROWS // 128
TEDGE = N_EDGES // NS
RROWS = NPAD * L // 128
RSLICE = RROWS // (NC * NS)
HTILE = HROWS // (NC * NS)
OROWS = N_NODES * F_OUT // 128

_mesh = plsc.VectorSubcoreMesh(core_axis_name="c", subcore_axis_name="s")

_sc_params = pltpu.CompilerParams()
for _field, _val in (("needs_layout_passes", False), ("use_tc_tiling_on_sc", False)):
    if _field in pltpu.CompilerParams.__dataclass_fields__:
        _sc_params = dataclasses.replace(_sc_params, **{_field: _val})


def _newton_rsqrt(d):
    i = plsc.bitcast(d, jnp.int32)
    i = jnp.full((L,), 0x5F3759DF, jnp.int32) - lax.shift_right_logical(i, 1)
    y = plsc.bitcast(i, jnp.float32)
    half_d = d * 0.5
    for _ in range(3):
        y = y * (1.5 - half_d * y * y)
    return y


@functools.partial(
    pl.kernel,
    mesh=_mesh,
    out_type=jax.ShapeDtypeStruct((RROWS, 128), jnp.float32),
    scratch_types=[
        pltpu.VMEM((TEDGE,), jnp.int32),
        pltpu.VMEM((HROWS, L), jnp.float32),
        pltpu.VMEM((NIOTA, 128), jnp.int32),
        pltpu.VMEM((HTILE, L), jnp.float32),
        pltpu.VMEM((L,), jnp.float32),
        pltpu.VMEM((2 * HTILE, 128), jnp.float32),
        pltpu.VMEM_SHARED((HROWS, L), jnp.float32),
    ],
    compiler_params=_sc_params,
)
def _hist_kernel(e_hbm, iota_hbm, out_hbm, dst_v, hist_v, iota_v, deg_v, dis_v,
                 rep_v, hist_sh):
    c = lax.axis_index("c")
    s = lax.axis_index("s")
    wid = c * NS + s
    pltpu.sync_copy(e_hbm.at[1, pl.ds(s * TEDGE, TEDGE)], dst_v)
    pltpu.sync_copy(iota_hbm, iota_v)

    zeros = jnp.zeros((L,), jnp.float32)

    @pl.loop(0, HROWS)
    def _(r):
        hist_v[r, :] = zeros

    pltpu.sync_copy(
        hist_v.at[pl.ds(s * HSLICE, HSLICE)],
        hist_sh.at[pl.ds(s * HSLICE, HSLICE)],
    )
    plsc.subcore_barrier()

    ones = jnp.ones((L,), jnp.float32)

    @pl.loop(0, TEDGE // L)
    def _(j):
        idx = dst_v[pl.ds(j * L, L)]
        row = lax.shift_right_logical(idx, 4)
        col = lax.bitwise_and(idx, 15)
        plsc.addupdate_scatter(hist_v, [row, col], ones)

    @pl.loop(0, NIOTA)
    def _(r):
        pltpu.sync_copy(
            hist_v.at[pl.ds(r * 128, 128)],
            hist_sh.at[iota_v.at[r]],
            add=True,
        )

    plsc.subcore_barrier()
    pltpu.sync_copy(hist_sh.at[pl.ds(wid * HTILE, HTILE)], deg_v)

    kvec = lax.iota(jnp.int32, L)
    rowoff = lax.shift_right_logical(kvec, 3)
    colbase = lax.shift_left(lax.bitwise_and(kvec, 7), 4)

    @pl.loop(0, HTILE)
    def _(r):
        dis = _newton_rsqrt(deg_v[r, :] + 1.0)
        dis_v[...] = dis
        rowv = rowoff + 2 * r
        for m in range(L):
            plsc.store_scatter(rep_v, [rowv, colbase + m], dis)

    pltpu.sync_copy(rep_v, out_hbm.at[pl.ds(wid * RSLICE, RSLICE)])


@functools.partial(
    pl.kernel,
    mesh=_mesh,
    out_type=jax.ShapeDtypeStruct((NC, N_NODES, F_OUT), jnp.float32),
    scratch_types=[
        pltpu.VMEM((NMMAX * KW,), jnp.int32),
        pltpu.VMEM((NMMAX * KW,), jnp.int32),
        pltpu.VMEM((KW, F_OUT), jnp.float32),
        pltpu.VMEM_SHARED((N_NODES, F_OUT), jnp.float32),
    ],
    compiler_params=_sc_params,
)
def _gather_scatter_kernel(h2_hbm, e_hbm, zeros_hbm, out_hbm, src_v, dst_v,
                           msg_v, acc_sh):
    c = lax.axis_index("c")
    s = lax.axis_index("s")
    pltpu.sync_copy(
        zeros_hbm.at[pl.ds(s * ASLICE, ASLICE)],
        acc_sh.at[pl.ds(s * ASLICE, ASLICE)],
    )

    for row, idx_v in ((0, src_v), (1, dst_v)):
        @pl.when(c == 0)
        def _():
            pltpu.sync_copy(
                e_hbm.at[row, pl.ds(s * (NB0 * 128), NB0 * 128)],
                idx_v.at[pl.ds(0, NB0 * 128)],
            )

        @pl.when(c == 1)
        def _():
            pltpu.sync_copy(
                e_hbm.at[row, pl.ds(E0 + s * (NB1 * 128), NB1 * 128)],
                idx_v.at[pl.ds(0, NB1 * 128)],
            )

        @pl.when((c == 0) & (s < NREM))
        def _():
            pltpu.sync_copy(
                e_hbm.at[row, pl.ds(E0 + E1 + s * 128, 128)],
                idx_v.at[pl.ds(NM0 * KW, 128)],
            )

    plsc.subcore_barrier()

    nmac = jnp.where(c == 0, NM0, NM1).astype(jnp.int32)

    @pl.loop(0, nmac)
    def _(j):
        pltpu.sync_copy(h2_hbm.at[src_v.at[pl.ds(j * KW, KW)]], msg_v)
        pltpu.sync_copy(msg_v, acc_sh.at[dst_v.at[pl.ds(j * KW, KW)]], add=True)

    @pl.when((c == 0) & (s < NREM))
    def _():
        pltpu.sync_copy(
            h2_hbm.at[src_v.at[pl.ds(NM0 * KW, 128)]],
            msg_v.at[pl.ds(0, 128)],
        )
        pltpu.sync_copy(
            msg_v.at[pl.ds(0, 128)],
            acc_sh.at[dst_v.at[pl.ds(NM0 * KW, 128)]],
            add=True,
        )

    plsc.subcore_barrier()
    pltpu.sync_copy(
        acc_sh.at[pl.ds(s * ASLICE, ASLICE)],
        out_hbm.at[c, pl.ds(s * ASLICE, ASLICE)],
    )


def _matmul_body(x_ref, w_ref, o_ref):
    o_ref[...] = jnp.dot(x_ref[...], w_ref[...], preferred_element_type=jnp.float32)


def _scale_body(h_ref, rep_ref, h2_ref):
    h2_ref[...] = h_ref[...] * rep_ref[...]


def _final_body(acc_ref, h2_ref, rep_ref, b_ref, o_ref):
    acc = acc_ref[0] + acc_ref[1]
    o_ref[...] = (acc + h2_ref[...]) * rep_ref[...] + b_ref[...]


def kernel(x, edge_index, W, b):
    iota = jnp.arange(HROWS, dtype=jnp.int32).reshape(NIOTA, 128)
    zeros = jnp.zeros((N_NODES, F_OUT), jnp.float32)
    dis_rep = _hist_kernel(edge_index, iota)
    rep16 = dis_rep.reshape(NPAD, F_OUT)[:N_NODES]
    h = pl.pallas_call(
        _matmul_body,
        out_shape=jax.ShapeDtypeStruct((N_NODES, F_OUT), jnp.float32),
    )(x, W)

    h2 = pl.pallas_call(
        _scale_body,
        out_shape=jax.ShapeDtypeStruct((N_NODES, F_OUT), jnp.float32),
    )(h, rep16)

    acc = _gather_scatter_kernel(h2, edge_index, zeros)

    out = pl.pallas_call(
        _final_body,
        out_shape=jax.ShapeDtypeStruct((N_NODES, F_OUT), jnp.float32),
    )(acc, h2, rep16, b.reshape(1, F_OUT))

    return out


# speedup vs baseline: 80.3853x; 1.2570x over previous
import dataclasses
import functools

import jax
import jax.numpy as jnp
from jax import lax
from jax.experimental import pallas as pl
from jax.experimental.pallas import tpu as pltpu
from jax.experimental.pallas import tpu_sc as plsc

N_NODES = 10000
N_EDGES = 320000
F_IN = 128
F_OUT = 16

NC = 2
NS = 16
L = 16

NB0 = 80
NB1 = 76
NREM = 4
E0 = NS * NB0 * 128
E1 = NS * NB1 * 128
KI = 4
KW = KI * 128
NM0 = NB0 // KI
NM1 = NB1 // KI
NMMAX = NM0 + 1

NPAD = 10240
HROWS = NPAD // L
HSLICE = HROWS // NS
ASLICE = N_NODES // NS
NIOTA = HROWS // 128
TEDGE = N_EDGES // NS
RROWS = NPAD * L // 128
RSLICE = RROWS // (NC * NS)
HTILE = HROWS // (NC * NS)
OROWS = N_NODES * F_OUT // 128

_mesh = plsc.VectorSubcoreMesh(core_axis_name="c", subcore_axis_name="s")

_sc_params = pltpu.CompilerParams()
for _field, _val in (("needs_layout_passes", False), ("use_tc_tiling_on_sc", False)):
    if _field in pltpu.CompilerParams.__dataclass_fields__:
        _sc_params = dataclasses.replace(_sc_params, **{_field: _val})


def _newton_rsqrt(d):
    i = plsc.bitcast(d, jnp.int32)
    i = jnp.full((L,), 0x5F3759DF, jnp.int32) - lax.shift_right_logical(i, 1)
    y = plsc.bitcast(i, jnp.float32)
    half_d = d * 0.5
    for _ in range(3):
        y = y * (1.5 - half_d * y * y)
    return y


@functools.partial(
    pl.kernel,
    mesh=_mesh,
    out_type=jax.ShapeDtypeStruct((RROWS, 128), jnp.float32),
    scratch_types=[
        pltpu.VMEM((TEDGE,), jnp.int32),
        pltpu.VMEM((HROWS, L), jnp.float32),
        pltpu.VMEM((NIOTA, 128), jnp.int32),
        pltpu.VMEM((HTILE, L), jnp.float32),
        pltpu.VMEM((L,), jnp.float32),
        pltpu.VMEM((2 * HTILE, 128), jnp.float32),
        pltpu.VMEM_SHARED((HROWS, L), jnp.float32),
    ],
    compiler_params=_sc_params,
)
def _hist_kernel(e_hbm, iota_hbm, out_hbm, dst_v, hist_v, iota_v, deg_v, dis_v,
                 rep_v, hist_sh):
    c = lax.axis_index("c")
    s = lax.axis_index("s")
    wid = c * NS + s
    pltpu.sync_copy(e_hbm.at[1, pl.ds(s * TEDGE, TEDGE)], dst_v)
    pltpu.sync_copy(iota_hbm, iota_v)

    zeros = jnp.zeros((L,), jnp.float32)

    @pl.loop(0, HROWS)
    def _(r):
        hist_v[r, :] = zeros

    pltpu.sync_copy(
        hist_v.at[pl.ds(s * HSLICE, HSLICE)],
        hist_sh.at[pl.ds(s * HSLICE, HSLICE)],
    )
    plsc.subcore_barrier()

    ones = jnp.ones((L,), jnp.float32)

    @pl.loop(0, TEDGE // L)
    def _(j):
        idx = dst_v[pl.ds(j * L, L)]
        row = lax.shift_right_logical(idx, 4)
        col = lax.bitwise_and(idx, 15)
        plsc.addupdate_scatter(hist_v, [row, col], ones)

    @pl.loop(0, NIOTA)
    def _(r):
        pltpu.sync_copy(
            hist_v.at[pl.ds(r * 128, 128)],
            hist_sh.at[iota_v.at[r]],
            add=True,
        )

    plsc.subcore_barrier()
    pltpu.sync_copy(hist_sh.at[pl.ds(wid * HTILE, HTILE)], deg_v)

    kvec = lax.iota(jnp.int32, L)
    rowoff = lax.shift_right_logical(kvec, 3)
    colbase = lax.shift_left(lax.bitwise_and(kvec, 7), 4)

    @pl.loop(0, HTILE)
    def _(r):
        dis = _newton_rsqrt(deg_v[r, :] + 1.0)
        dis_v[...] = dis
        rowv = rowoff + 2 * r
        for m in range(L):
            plsc.store_scatter(rep_v, [rowv, colbase + m], dis)

    pltpu.sync_copy(rep_v, out_hbm.at[pl.ds(wid * RSLICE, RSLICE)])


@functools.partial(
    pl.kernel,
    mesh=_mesh,
    out_type=jax.ShapeDtypeStruct((NC, N_NODES, F_OUT), jnp.float32),
    scratch_types=[
        pltpu.VMEM((NMMAX * KW,), jnp.int32),
        pltpu.VMEM((NMMAX * KW,), jnp.int32),
        pltpu.VMEM((KW, F_OUT), jnp.float32),
        pltpu.VMEM_SHARED((N_NODES, F_OUT), jnp.float32),
    ],
    compiler_params=_sc_params,
)
def _gather_scatter_kernel(h2_hbm, e_hbm, zeros_hbm, out_hbm, src_v, dst_v,
                           msg_v, acc_sh):
    c = lax.axis_index("c")
    s = lax.axis_index("s")
    pltpu.sync_copy(
        zeros_hbm.at[pl.ds(s * ASLICE, ASLICE)],
        acc_sh.at[pl.ds(s * ASLICE, ASLICE)],
    )

    for row, idx_v in ((0, src_v), (1, dst_v)):
        @pl.when(c == 0)
        def _():
            pltpu.sync_copy(
                e_hbm.at[row, pl.ds(s * (NB0 * 128), NB0 * 128)],
                idx_v.at[pl.ds(0, NB0 * 128)],
            )

        @pl.when(c == 1)
        def _():
            pltpu.sync_copy(
                e_hbm.at[row, pl.ds(E0 + s * (NB1 * 128), NB1 * 128)],
                idx_v.at[pl.ds(0, NB1 * 128)],
            )

        @pl.when((c == 0) & (s < NREM))
        def _():
            pltpu.sync_copy(
                e_hbm.at[row, pl.ds(E0 + E1 + s * 128, 128)],
                idx_v.at[pl.ds(NM0 * KW, 128)],
            )

    plsc.subcore_barrier()

    nmac = jnp.where(c == 0, NM0, NM1).astype(jnp.int32)

    @pl.loop(0, nmac)
    def _(j):
        pltpu.sync_copy(h2_hbm.at[src_v.at[pl.ds(j * KW, KW)]], msg_v)
        pltpu.sync_copy(msg_v, acc_sh.at[dst_v.at[pl.ds(j * KW, KW)]], add=True)

    @pl.when((c == 0) & (s < NREM))
    def _():
        pltpu.sync_copy(
            h2_hbm.at[src_v.at[pl.ds(NM0 * KW, 128)]],
            msg_v.at[pl.ds(0, 128)],
        )
        pltpu.sync_copy(
            msg_v.at[pl.ds(0, 128)],
            acc_sh.at[dst_v.at[pl.ds(NM0 * KW, 128)]],
            add=True,
        )

    plsc.subcore_barrier()
    pltpu.sync_copy(
        acc_sh.at[pl.ds(s * ASLICE, ASLICE)],
        out_hbm.at[c, pl.ds(s * ASLICE, ASLICE)],
    )


def _matmul_body(x8_ref, w_ref, o_ref):
    w = w_ref[...]
    for k in range(8):
        o_ref[:, pl.ds(k * F_OUT, F_OUT)] = jnp.dot(
            x8_ref[:, k, :], w, preferred_element_type=jnp.float32
        )


def _scale_body(h_ref, rep_ref, h2_ref):
    h2_ref[...] = h_ref[...] * rep_ref[pl.ds(0, OROWS), :]


def _final_body(acc_ref, h2_ref, rep_ref, b_ref, o_ref):
    acc = acc_ref[0] + acc_ref[1]
    o_ref[...] = (acc + h2_ref[...]) * rep_ref[pl.ds(0, OROWS), :] + b_ref[...]


def kernel(x, edge_index, W, b):
    iota = jnp.arange(HROWS, dtype=jnp.int32).reshape(NIOTA, 128)
    zeros = jnp.zeros((N_NODES, F_OUT), jnp.float32)
    b128 = jnp.tile(b, 8).reshape(1, 128)
    x8 = x.reshape(OROWS, 8, F_IN)

    dis_rep = _hist_kernel(edge_index, iota)
    h128 = pl.pallas_call(
        _matmul_body,
        out_shape=jax.ShapeDtypeStruct((OROWS, 128), jnp.float32),
    )(x8, W)

    h2_128 = pl.pallas_call(
        _scale_body,
        out_shape=jax.ShapeDtypeStruct((OROWS, 128), jnp.float32),
    )(h128, dis_rep)
    h2 = h2_128.reshape(N_NODES, F_OUT)

    acc = _gather_scatter_kernel(h2, edge_index, zeros)
    acc128 = acc.reshape(NC, OROWS, 128)

    out128 = pl.pallas_call(
        _final_body,
        out_shape=jax.ShapeDtypeStruct((OROWS, 128), jnp.float32),
    )(acc128, h2_128, dis_rep, b128)

    return out128.reshape(N_NODES, F_OUT)


# speedup vs baseline: 85.6270x; 1.0652x over previous
import dataclasses
import functools

import jax
import jax.numpy as jnp
from jax import lax
from jax.experimental import pallas as pl
from jax.experimental.pallas import tpu as pltpu
from jax.experimental.pallas import tpu_sc as plsc

N_NODES = 10000
N_EDGES = 320000
F_IN = 128
F_OUT = 16

NC = 2
NS = 16
L = 16

NB0 = 84
NB1 = 72
NREM = 4
E0 = NS * NB0 * 128
E1 = NS * NB1 * 128
KW = 1024
NM0 = 10
NM1 = 9
IDXN = NB0 * 128 + 128

NPAD = 10240
HROWS = NPAD // L
HSLICE = HROWS // NS
ASLICE = N_NODES // NS
NIOTA = HROWS // 128
TEDGE = N_EDGES // NS
RROWS = NPAD * L // 128
RSLICE = RROWS // (NC * NS)
HTILE = HROWS // (NC * NS)
OROWS = N_NODES * F_OUT // 128

_mesh = plsc.VectorSubcoreMesh(core_axis_name="c", subcore_axis_name="s")

_sc_params = pltpu.CompilerParams()
for _field, _val in (("needs_layout_passes", False), ("use_tc_tiling_on_sc", False)):
    if _field in pltpu.CompilerParams.__dataclass_fields__:
        _sc_params = dataclasses.replace(_sc_params, **{_field: _val})


def _newton_rsqrt(d):
    i = plsc.bitcast(d, jnp.int32)
    i = jnp.full((L,), 0x5F3759DF, jnp.int32) - lax.shift_right_logical(i, 1)
    y = plsc.bitcast(i, jnp.float32)
    half_d = d * 0.5
    for _ in range(3):
        y = y * (1.5 - half_d * y * y)
    return y


@functools.partial(
    pl.kernel,
    mesh=_mesh,
    out_type=jax.ShapeDtypeStruct((RROWS, 128), jnp.float32),
    scratch_types=[
        pltpu.VMEM((TEDGE,), jnp.int32),
        pltpu.VMEM((HROWS, L), jnp.float32),
        pltpu.VMEM((NIOTA, 128), jnp.int32),
        pltpu.VMEM((HTILE, L), jnp.float32),
        pltpu.VMEM((L,), jnp.float32),
        pltpu.VMEM((2 * HTILE, 128), jnp.float32),
        pltpu.VMEM_SHARED((HROWS, L), jnp.float32),
    ],
    compiler_params=_sc_params,
)
def _hist_kernel(e_hbm, iota_hbm, out_hbm, dst_v, hist_v, iota_v, deg_v, dis_v,
                 rep_v, hist_sh):
    c = lax.axis_index("c")
    s = lax.axis_index("s")
    wid = c * NS + s
    pltpu.sync_copy(e_hbm.at[1, pl.ds(s * TEDGE, TEDGE)], dst_v)
    pltpu.sync_copy(iota_hbm, iota_v)

    zeros = jnp.zeros((L,), jnp.float32)

    @pl.loop(0, HROWS)
    def _(r):
        hist_v[r, :] = zeros

    pltpu.sync_copy(
        hist_v.at[pl.ds(s * HSLICE, HSLICE)],
        hist_sh.at[pl.ds(s * HSLICE, HSLICE)],
    )
    plsc.subcore_barrier()

    ones = jnp.ones((L,), jnp.float32)

    @pl.loop(0, TEDGE // L)
    def _(j):
        idx = dst_v[pl.ds(j * L, L)]
        row = lax.shift_right_logical(idx, 4)
        col = lax.bitwise_and(idx, 15)
        plsc.addupdate_scatter(hist_v, [row, col], ones)

    @pl.loop(0, NIOTA)
    def _(r):
        pltpu.sync_copy(
            hist_v.at[pl.ds(r * 128, 128)],
            hist_sh.at[iota_v.at[r]],
            add=True,
        )

    plsc.subcore_barrier()
    pltpu.sync_copy(hist_sh.at[pl.ds(wid * HTILE, HTILE)], deg_v)

    kvec = lax.iota(jnp.int32, L)
    rowoff = lax.shift_right_logical(kvec, 3)
    colbase = lax.shift_left(lax.bitwise_and(kvec, 7), 4)

    @pl.loop(0, HTILE)
    def _(r):
        dis = _newton_rsqrt(deg_v[r, :] + 1.0)
        dis_v[...] = dis
        rowv = rowoff + 2 * r
        for m in range(L):
            plsc.store_scatter(rep_v, [rowv, colbase + m], dis)

    pltpu.sync_copy(rep_v, out_hbm.at[pl.ds(wid * RSLICE, RSLICE)])


@functools.partial(
    pl.kernel,
    mesh=_mesh,
    out_type=jax.ShapeDtypeStruct((NC, N_NODES, F_OUT), jnp.float32),
    scratch_types=[
        pltpu.VMEM((IDXN,), jnp.int32),
        pltpu.VMEM((IDXN,), jnp.int32),
        pltpu.VMEM((KW, F_OUT), jnp.float32),
        pltpu.VMEM_SHARED((N_NODES, F_OUT), jnp.float32),
    ],
    compiler_params=_sc_params,
)
def _gather_scatter_kernel(h2_hbm, e_hbm, zeros_hbm, out_hbm, src_v, dst_v,
                           msg_v, acc_sh):
    c = lax.axis_index("c")
    s = lax.axis_index("s")
    pltpu.sync_copy(
        zeros_hbm.at[pl.ds(s * ASLICE, ASLICE)],
        acc_sh.at[pl.ds(s * ASLICE, ASLICE)],
    )

    for row, idx_v in ((0, src_v), (1, dst_v)):
        @pl.when(c == 0)
        def _():
            pltpu.sync_copy(
                e_hbm.at[row, pl.ds(s * (NB0 * 128), NB0 * 128)],
                idx_v.at[pl.ds(0, NB0 * 128)],
            )

        @pl.when(c == 1)
        def _():
            pltpu.sync_copy(
                e_hbm.at[row, pl.ds(E0 + s * (NB1 * 128), NB1 * 128)],
                idx_v.at[pl.ds(0, NB1 * 128)],
            )

        @pl.when((c == 0) & (s < NREM))
        def _():
            pltpu.sync_copy(
                e_hbm.at[row, pl.ds(E0 + E1 + s * 128, 128)],
                idx_v.at[pl.ds(NB0 * 128, 128)],
            )

    plsc.subcore_barrier()

    nmac = jnp.where(c == 0, NM0, NM1).astype(jnp.int32)

    def _move(off, n):
        pltpu.sync_copy(h2_hbm.at[src_v.at[pl.ds(off, n)]], msg_v.at[pl.ds(0, n)])
        pltpu.sync_copy(
            msg_v.at[pl.ds(0, n)], acc_sh.at[dst_v.at[pl.ds(off, n)]], add=True
        )

    @pl.loop(0, nmac)
    def _(j):
        _move(j * KW, KW)

    @pl.when(c == 0)
    def _():
        _move(NM0 * KW, 512)

    @pl.when((c == 0) & (s < NREM))
    def _():
        _move(NB0 * 128, 128)

    plsc.subcore_barrier()
    pltpu.sync_copy(
        acc_sh.at[pl.ds(s * ASLICE, ASLICE)],
        out_hbm.at[c, pl.ds(s * ASLICE, ASLICE)],
    )


def _matmul_body(x8_ref, w_ref, o_ref):
    w = w_ref[...]
    for k in range(8):
        o_ref[:, pl.ds(k * F_OUT, F_OUT)] = jnp.dot(
            x8_ref[:, k, :], w, preferred_element_type=jnp.float32
        )


def _scale_body(h_ref, rep_ref, h2_ref):
    h2_ref[...] = h_ref[...] * rep_ref[pl.ds(0, OROWS), :]


def _final_body(acc_ref, h2_ref, rep_ref, b_ref, o_ref):
    acc = acc_ref[0] + acc_ref[1]
    o_ref[...] = (acc + h2_ref[...]) * rep_ref[pl.ds(0, OROWS), :] + b_ref[...]


def kernel(x, edge_index, W, b):
    iota = jnp.arange(HROWS, dtype=jnp.int32).reshape(NIOTA, 128)
    zeros = jnp.zeros((N_NODES, F_OUT), jnp.float32)
    b128 = jnp.tile(b, 8).reshape(1, 128)
    x8 = x.reshape(OROWS, 8, F_IN)

    dis_rep = _hist_kernel(edge_index, iota)
    h128 = pl.pallas_call(
        _matmul_body,
        out_shape=jax.ShapeDtypeStruct((OROWS, 128), jnp.float32),
    )(x8, W)

    h2_128 = pl.pallas_call(
        _scale_body,
        out_shape=jax.ShapeDtypeStruct((OROWS, 128), jnp.float32),
    )(h128, dis_rep)
    h2 = h2_128.reshape(N_NODES, F_OUT)

    acc = _gather_scatter_kernel(h2, edge_index, zeros)
    acc128 = acc.reshape(NC, OROWS, 128)

    out128 = pl.pallas_call(
        _final_body,
        out_shape=jax.ShapeDtypeStruct((OROWS, 128), jnp.float32),
    )(acc128, h2_128, dis_rep, b128)

    return out128.reshape(N_NODES, F_OUT)


# speedup vs baseline: 97.3975x; 1.1375x over previous
import dataclasses
import functools

import jax
import jax.numpy as jnp
from jax import lax
from jax.experimental import pallas as pl
from jax.experimental.pallas import tpu as pltpu
from jax.experimental.pallas import tpu_sc as plsc

N_NODES = 10000
N_EDGES = 320000
F_IN = 128
F_OUT = 16

NC = 2
NS = 16
L = 16

NB0 = 84
NB1 = 72
NREM = 4
E0 = NS * NB0 * 128
E1 = NS * NB1 * 128
KW = 1024
NM0 = 10
NM1 = 9
IDXN = NB0 * 128 + 128

NPAD = 10240
HROWS = NPAD // L
HSLICE = HROWS // NS
ASLICE = N_NODES // NS
NIOTA = HROWS // 128
TEDGE = N_EDGES // NS
RROWS = NPAD * L // 128
RSLICE = RROWS // (NC * NS)
HTILE = HROWS // (NC * NS)
OROWS = N_NODES * F_OUT // 128

_mesh = plsc.VectorSubcoreMesh(core_axis_name="c", subcore_axis_name="s")

_sc_params = pltpu.CompilerParams()
for _field, _val in (("needs_layout_passes", False), ("use_tc_tiling_on_sc", False)):
    if _field in pltpu.CompilerParams.__dataclass_fields__:
        _sc_params = dataclasses.replace(_sc_params, **{_field: _val})


def _newton_rsqrt(d):
    i = plsc.bitcast(d, jnp.int32)
    i = jnp.full((L,), 0x5F3759DF, jnp.int32) - lax.shift_right_logical(i, 1)
    y = plsc.bitcast(i, jnp.float32)
    half_d = d * 0.5
    for _ in range(3):
        y = y * (1.5 - half_d * y * y)
    return y


@functools.partial(
    pl.kernel,
    mesh=_mesh,
    out_type=jax.ShapeDtypeStruct((RROWS, 128), jnp.float32),
    scratch_types=[
        pltpu.VMEM((TEDGE,), jnp.int32),
        pltpu.VMEM((HROWS, L), jnp.float32),
        pltpu.VMEM((NIOTA, 128), jnp.int32),
        pltpu.VMEM((HTILE, L), jnp.float32),
        pltpu.VMEM((L,), jnp.float32),
        pltpu.VMEM((2 * HTILE, 128), jnp.float32),
        pltpu.VMEM_SHARED((HROWS, L), jnp.float32),
    ],
    compiler_params=_sc_params,
)
def _hist_kernel(e_hbm, iota_hbm, out_hbm, dst_v, hist_v, iota_v, deg_v, dis_v,
                 rep_v, hist_sh):
    c = lax.axis_index("c")
    s = lax.axis_index("s")
    wid = c * NS + s
    pltpu.sync_copy(e_hbm.at[1, pl.ds(s * TEDGE, TEDGE)], dst_v)
    pltpu.sync_copy(iota_hbm, iota_v)

    zeros = jnp.zeros((L,), jnp.float32)

    @pl.loop(0, HROWS)
    def _(r):
        hist_v[r, :] = zeros

    pltpu.sync_copy(
        hist_v.at[pl.ds(s * HSLICE, HSLICE)],
        hist_sh.at[pl.ds(s * HSLICE, HSLICE)],
    )
    plsc.subcore_barrier()

    ones = jnp.ones((L,), jnp.float32)

    @pl.loop(0, TEDGE // L)
    def _(j):
        idx = dst_v[pl.ds(j * L, L)]
        row = lax.shift_right_logical(idx, 4)
        col = lax.bitwise_and(idx, 15)
        plsc.addupdate_scatter(hist_v, [row, col], ones)

    @pl.loop(0, NIOTA)
    def _(r):
        pltpu.sync_copy(
            hist_v.at[pl.ds(r * 128, 128)],
            hist_sh.at[iota_v.at[r]],
            add=True,
        )

    plsc.subcore_barrier()
    pltpu.sync_copy(hist_sh.at[pl.ds(wid * HTILE, HTILE)], deg_v)

    kvec = lax.iota(jnp.int32, L)
    rowoff = lax.shift_right_logical(kvec, 3)
    colbase = lax.shift_left(lax.bitwise_and(kvec, 7), 4)

    @pl.loop(0, HTILE)
    def _(r):
        dis = _newton_rsqrt(deg_v[r, :] + 1.0)
        dis_v[...] = dis
        rowv = rowoff + 2 * r
        for m in range(L):
            plsc.store_scatter(rep_v, [rowv, colbase + m], dis)

    pltpu.sync_copy(rep_v, out_hbm.at[pl.ds(wid * RSLICE, RSLICE)])


@functools.partial(
    pl.kernel,
    mesh=_mesh,
    out_type=jax.ShapeDtypeStruct((NC, N_NODES, F_OUT), jnp.float32),
    scratch_types=[
        pltpu.VMEM((IDXN,), jnp.int32),
        pltpu.VMEM((IDXN,), jnp.int32),
        pltpu.VMEM((KW, F_OUT), jnp.float32),
        pltpu.VMEM((KW, F_OUT), jnp.float32),
        pltpu.SemaphoreType.DMA,
        pltpu.SemaphoreType.DMA,
        pltpu.SemaphoreType.DMA,
        pltpu.SemaphoreType.DMA,
        pltpu.VMEM_SHARED((N_NODES, F_OUT), jnp.float32),
    ],
    compiler_params=_sc_params,
)
def _gather_scatter_kernel(h2_hbm, e_hbm, zeros_hbm, out_hbm, src_v, dst_v,
                           msg_a, msg_b, gsem_a, gsem_b, ssem_a, ssem_b, acc_sh):
    c = lax.axis_index("c")
    s = lax.axis_index("s")
    pltpu.sync_copy(
        zeros_hbm.at[pl.ds(s * ASLICE, ASLICE)],
        acc_sh.at[pl.ds(s * ASLICE, ASLICE)],
    )

    for row, idx_v in ((0, src_v), (1, dst_v)):
        @pl.when(c == 0)
        def _():
            pltpu.sync_copy(
                e_hbm.at[row, pl.ds(s * (NB0 * 128), NB0 * 128)],
                idx_v.at[pl.ds(0, NB0 * 128)],
            )

        @pl.when(c == 1)
        def _():
            pltpu.sync_copy(
                e_hbm.at[row, pl.ds(E0 + s * (NB1 * 128), NB1 * 128)],
                idx_v.at[pl.ds(0, NB1 * 128)],
            )

        @pl.when((c == 0) & (s < NREM))
        def _():
            pltpu.sync_copy(
                e_hbm.at[row, pl.ds(E0 + E1 + s * 128, 128)],
                idx_v.at[pl.ds(NB0 * 128, 128)],
            )

    plsc.subcore_barrier()

    msg = (msg_a, msg_b)
    gsem = (gsem_a, gsem_b)
    ssem = (ssem_a, ssem_b)

    def _gather(j):
        b = j % 2
        return pltpu.async_copy(
            h2_hbm.at[src_v.at[pl.ds(j * KW, KW)]], msg[b], gsem[b]
        )

    def _pipeline(n):
        gd = {0: _gather(0)}
        sd = {}
        for j in range(n):
            if j + 1 < n:
                if j - 1 >= 0:
                    sd[j - 1].wait()
                gd[j + 1] = _gather(j + 1)
            gd[j].wait()
            sd[j] = pltpu.async_copy(
                msg[j % 2], acc_sh.at[dst_v.at[pl.ds(j * KW, KW)]], ssem[j % 2],
                add=True,
            )
        if n >= 2:
            sd[n - 2].wait()
        sd[n - 1].wait()

    def _move(off, n):
        pltpu.sync_copy(h2_hbm.at[src_v.at[pl.ds(off, n)]], msg_a.at[pl.ds(0, n)])
        pltpu.sync_copy(
            msg_a.at[pl.ds(0, n)], acc_sh.at[dst_v.at[pl.ds(off, n)]], add=True
        )

    @pl.when(c == 0)
    def _():
        _pipeline(NM0)
        _move(NM0 * KW, 512)

    @pl.when(c == 1)
    def _():
        _pipeline(NM1)

    @pl.when((c == 0) & (s < NREM))
    def _():
        _move(NB0 * 128, 128)

    plsc.subcore_barrier()
    pltpu.sync_copy(
        acc_sh.at[pl.ds(s * ASLICE, ASLICE)],
        out_hbm.at[c, pl.ds(s * ASLICE, ASLICE)],
    )


def _matmul_body(x8_ref, w_ref, o_ref):
    w = w_ref[...]
    for k in range(8):
        o_ref[:, pl.ds(k * F_OUT, F_OUT)] = jnp.dot(
            x8_ref[:, k, :], w, preferred_element_type=jnp.float32
        )


def _scale_body(h_ref, rep_ref, h2_ref):
    h2_ref[...] = h_ref[...] * rep_ref[pl.ds(0, OROWS), :]


def _final_body(acc_ref, h2_ref, rep_ref, b_ref, o_ref):
    acc = acc_ref[0] + acc_ref[1]
    o_ref[...] = (acc + h2_ref[...]) * rep_ref[pl.ds(0, OROWS), :] + b_ref[...]


def kernel(x, edge_index, W, b):
    iota = jnp.arange(HROWS, dtype=jnp.int32).reshape(NIOTA, 128)
    zeros = jnp.zeros((N_NODES, F_OUT), jnp.float32)
    b128 = jnp.tile(b, 8).reshape(1, 128)
    x8 = x.reshape(OROWS, 8, F_IN)

    dis_rep = _hist_kernel(edge_index, iota)
    h128 = pl.pallas_call(
        _matmul_body,
        out_shape=jax.ShapeDtypeStruct((OROWS, 128), jnp.float32),
    )(x8, W)

    h2_128 = pl.pallas_call(
        _scale_body,
        out_shape=jax.ShapeDtypeStruct((OROWS, 128), jnp.float32),
    )(h128, dis_rep)
    h2 = h2_128.reshape(N_NODES, F_OUT)

    acc = _gather_scatter_kernel(h2, edge_index, zeros)
    acc128 = acc.reshape(NC, OROWS, 128)

    out128 = pl.pallas_call(
        _final_body,
        out_shape=jax.ShapeDtypeStruct((OROWS, 128), jnp.float32),
    )(acc128, h2_128, dis_rep, b128)

    return out128.reshape(N_NODES, F_OUT)
